# baseline (speedup 1.0000x reference)
import functools

import jax
import jax.numpy as jnp
from jax import lax
from jax.experimental import pallas as pl
from jax.experimental.pallas import tpu as pltpu
from jax.experimental.pallas import tpu_sc as plsc

_NUM_CORES = 2
_NUM_SUBCORES = 16
_NUM_WORKERS = _NUM_CORES * _NUM_SUBCORES
_LANES = 16

_VOCAB = 1000000
_BATCH = 4096
_SEQ = 200
_DIM = 64
_CHUNK = 256
_T_PER_W = _BATCH * _SEQ // _NUM_WORKERS
_B_PER_W = _BATCH // _NUM_WORKERS
_H_PER_W = _T_PER_W // _CHUNK
_NBUF = 4
_LAG = _NBUF // 2
_NVEC = _DIM // _LANES


def _make_sc_call():
    mesh = plsc.VectorSubcoreMesh(core_axis_name="c", subcore_axis_name="s")

    @functools.partial(
        pl.kernel,
        mesh=mesh,
        compiler_params=pltpu.CompilerParams(use_tc_tiling_on_sc=False),
        out_type=jax.ShapeDtypeStruct((_BATCH, _DIM), jnp.float32),
        scratch_types=[
            pltpu.VMEM((_H_PER_W, _CHUNK), jnp.int32),
            pltpu.VMEM((_NBUF, _CHUNK, _DIM), jnp.float32),
            pltpu.VMEM((_NBUF, _CHUNK), jnp.int32),
            pltpu.VMEM((_B_PER_W, _DIM), jnp.float32),
            pltpu.VMEM_SHARED((_NUM_SUBCORES * _B_PER_W, _DIM), jnp.float32),
            [pltpu.SemaphoreType.DMA] * _NBUF,
            [pltpu.SemaphoreType.DMA] * _NBUF,
        ],
    )
    def enc(tokens_hbm, table_hbm, out_hbm, idx_v, rows_v, sidx_v, out_v,
            acc_sh, gsems, ssems):
        cid = lax.axis_index("c")
        sid = lax.axis_index("s")
        wid = sid * _NUM_CORES + cid
        base_h = wid * _H_PER_W
        base_b = wid * _B_PER_W
        own = sid * _B_PER_W

        zvec = jnp.zeros((_LANES,), jnp.float32)

        def zbody(r, carry):
            for k in range(_NVEC):
                out_v[r, pl.ds(k * _LANES, _LANES)] = zvec
            return carry

        lax.fori_loop(0, _B_PER_W, zbody, 0)
        pltpu.sync_copy(out_v, acc_sh.at[pl.ds(own, _B_PER_W)])

        pltpu.make_async_copy(
            tokens_hbm.at[pl.ds(base_h, _H_PER_W)], idx_v, gsems[0]).start()
        pltpu.make_async_copy(
            tokens_hbm.at[pl.ds(base_h, _H_PER_W)], idx_v, gsems[0]).wait()

        def gather_start(h, buf):
            def gbody(g, carry):
                vec = idx_v[h, pl.ds(g * _LANES, _LANES)]
                for l in range(_LANES):
                    r = g * _LANES + l
                    pltpu.make_async_copy(
                        table_hbm.at[vec[l]],
                        rows_v.at[buf, r],
                        gsems[buf]).start()
                return carry
            lax.fori_loop(0, _CHUNK // _LANES, gbody, 0)

        def gather_wait(buf):
            pltpu.make_async_copy(
                table_hbm.at[pl.ds(0, _CHUNK)], rows_v.at[buf],
                gsems[buf]).wait()

        def scatter(buf):
            return pltpu.make_async_copy(
                rows_v.at[buf], acc_sh.at[sidx_v.at[buf]], ssems[buf])

        def scatter_start(buf):
            pltpu.async_copy(
                rows_v.at[buf], acc_sh.at[sidx_v.at[buf]], ssems[buf],
                add=True)

        iota = lax.iota(jnp.int32, _LANES)

        def set_scatter_rows(buf, h):
            pos0 = h * _CHUNK
            for g in range(_CHUNK // _LANES):
                pos = iota + (pos0 + g * _LANES)
                tgt = lax.div(pos, jnp.int32(_SEQ)) + own
                sidx_v[buf, pl.ds(g * _LANES, _LANES)] = tgt

        for b in range(_LAG):
            gather_start(jnp.int32(b), b)

        def outer(i, carry):
            for j in range(_NBUF):
                h = i * _NBUF + j
                gather_wait(j)
                set_scatter_rows(j, h)
                scatter_start(j)
                nb = (j + _LAG) % _NBUF
                nh = h + _LAG

                @pl.when(nh >= _NBUF)
                def _():
                    scatter(nb).wait()

                @pl.when(nh < _H_PER_W)
                def _():
                    gather_start(nh, nb)
            return carry

        lax.fori_loop(0, _H_PER_W // _NBUF, outer, 0)

        for j in range(_NBUF - _LAG, _NBUF):
            scatter(j).wait()

        pltpu.sync_copy(acc_sh.at[pl.ds(own, _B_PER_W)], out_v)
        inv_n = jnp.float32(1.0 / _SEQ)

        def scale(r, carry):
            for k in range(_NVEC):
                sl = pl.ds(k * _LANES, _LANES)
                out_v[r, sl] = out_v[r, sl] * inv_n
            return carry

        lax.fori_loop(0, _B_PER_W, scale, 0)
        pltpu.make_async_copy(
            out_v, out_hbm.at[pl.ds(base_b, _B_PER_W)], gsems[0]).start()
        pltpu.make_async_copy(
            out_v, out_hbm.at[pl.ds(base_b, _B_PER_W)], gsems[0]).wait()

    return enc


_sc_call = _make_sc_call()


def kernel(tokens, embedding_weight):
    tokens2 = tokens.reshape(_BATCH * _SEQ // _CHUNK, _CHUNK)
    return _sc_call(tokens2, embedding_weight)


# speedup vs baseline: 1.0003x; 1.0003x over previous
import functools

import jax
import jax.numpy as jnp
from jax import lax
from jax.experimental import pallas as pl
from jax.experimental.pallas import tpu as pltpu
from jax.experimental.pallas import tpu_sc as plsc

_NUM_CORES = 2
_NUM_SUBCORES = 16
_NUM_WORKERS = _NUM_CORES * _NUM_SUBCORES
_LANES = 16

_VOCAB = 1000000
_BATCH = 4096
_SEQ = 200
_DIM = 64
_CHUNK = _SEQ
_T_PER_W = _BATCH * _SEQ // _NUM_WORKERS
_B_PER_W = _BATCH // _NUM_WORKERS
_H_PER_W = _T_PER_W // _CHUNK
_NBUF = 4
_LAG = _NBUF // 2
_NVEC = _DIM // _LANES


def _make_sc_call():
    mesh = plsc.VectorSubcoreMesh(core_axis_name="c", subcore_axis_name="s")

    @functools.partial(
        pl.kernel,
        mesh=mesh,
        compiler_params=pltpu.CompilerParams(use_tc_tiling_on_sc=False),
        out_type=jax.ShapeDtypeStruct((_BATCH, _DIM), jnp.float32),
        scratch_types=[
            pltpu.VMEM((_H_PER_W, _CHUNK), jnp.int32),
            pltpu.VMEM((_NBUF, _CHUNK, _DIM), jnp.float32),
            pltpu.VMEM((_NBUF, _CHUNK), jnp.int32),
            pltpu.VMEM((_B_PER_W, _DIM), jnp.float32),
            pltpu.VMEM_SHARED((_NUM_SUBCORES * _B_PER_W, _DIM), jnp.float32),
            [pltpu.SemaphoreType.DMA] * _NBUF,
            [pltpu.SemaphoreType.DMA] * _NBUF,
        ],
    )
    def enc(tokens_hbm, table_hbm, out_hbm, idx_v, rows_v, sidx_v, out_v,
            acc_sh, gsems, ssems):
        cid = lax.axis_index("c")
        sid = lax.axis_index("s")
        wid = sid * _NUM_CORES + cid
        base_h = wid * _H_PER_W
        base_b = wid * _B_PER_W
        own = sid * _B_PER_W

        zvec = jnp.zeros((_LANES,), jnp.float32)

        def zbody(r, carry):
            for k in range(_NVEC):
                out_v[r, pl.ds(k * _LANES, _LANES)] = zvec
            return carry

        lax.fori_loop(0, _B_PER_W, zbody, 0)
        pltpu.sync_copy(out_v, acc_sh.at[pl.ds(own, _B_PER_W)])

        pltpu.make_async_copy(
            tokens_hbm.at[pl.ds(base_h, _H_PER_W)], idx_v, gsems[0]).start()
        pltpu.make_async_copy(
            tokens_hbm.at[pl.ds(base_h, _H_PER_W)], idx_v, gsems[0]).wait()

        def gather_start(h, buf):
            def gbody(g, carry):
                vec = idx_v[h, pl.ds(g * _LANES, _LANES)]
                for l in range(_LANES):
                    r = g * _LANES + l
                    pltpu.make_async_copy(
                        table_hbm.at[vec[l]],
                        rows_v.at[buf, r],
                        gsems[buf]).start()
                return carry
            lax.fori_loop(0, 12, gbody, 0)
            vec = idx_v[h, pl.ds(_CHUNK - _LANES, _LANES)]
            for l in range(8, _LANES):
                r = _CHUNK - _LANES + l
                pltpu.make_async_copy(
                    table_hbm.at[vec[l]],
                    rows_v.at[buf, r],
                    gsems[buf]).start()

        def gather_wait(buf):
            pltpu.make_async_copy(
                table_hbm.at[pl.ds(0, _CHUNK)], rows_v.at[buf],
                gsems[buf]).wait()

        def scatter(buf):
            return pltpu.make_async_copy(
                rows_v.at[buf], acc_sh.at[sidx_v.at[buf]], ssems[buf])

        def scatter_start(buf):
            pltpu.async_copy(
                rows_v.at[buf], acc_sh.at[sidx_v.at[buf]], ssems[buf],
                add=True)

        _splat_offs = tuple(range(0, _CHUNK - _LANES, _LANES)) + (_CHUNK - _LANES,)

        def set_scatter_rows(buf, h):
            val = jnp.zeros((_LANES,), jnp.int32) + (own + h)
            for off in _splat_offs:
                sidx_v[buf, pl.ds(off, _LANES)] = val

        for b in range(_LAG):
            gather_start(jnp.int32(b), b)

        def outer(i, carry):
            for j in range(_NBUF):
                h = i * _NBUF + j
                gather_wait(j)
                set_scatter_rows(j, h)
                scatter_start(j)
                nb = (j + _LAG) % _NBUF
                nh = h + _LAG

                @pl.when(nh >= _NBUF)
                def _():
                    scatter(nb).wait()

                @pl.when(nh < _H_PER_W)
                def _():
                    gather_start(nh, nb)
            return carry

        lax.fori_loop(0, _H_PER_W // _NBUF, outer, 0)

        for j in range(_NBUF - _LAG, _NBUF):
            scatter(j).wait()

        pltpu.sync_copy(acc_sh.at[pl.ds(own, _B_PER_W)], out_v)
        inv_n = jnp.float32(1.0 / _SEQ)

        def scale(r, carry):
            for k in range(_NVEC):
                sl = pl.ds(k * _LANES, _LANES)
                out_v[r, sl] = out_v[r, sl] * inv_n
            return carry

        lax.fori_loop(0, _B_PER_W, scale, 0)
        pltpu.make_async_copy(
            out_v, out_hbm.at[pl.ds(base_b, _B_PER_W)], gsems[0]).start()
        pltpu.make_async_copy(
            out_v, out_hbm.at[pl.ds(base_b, _B_PER_W)], gsems[0]).wait()

    return enc


_sc_call = _make_sc_call()


def kernel(tokens, embedding_weight):
    return _sc_call(tokens, embedding_weight)


# speedup vs baseline: 1.4815x; 1.4810x over previous
import functools

import jax
import jax.numpy as jnp
from jax import lax
from jax.experimental import pallas as pl
from jax.experimental.pallas import tpu as pltpu
from jax.experimental.pallas import tpu_sc as plsc
from jax.experimental import layout as jex_layout

_NUM_CORES = 2
_NUM_SUBCORES = 16
_NUM_WORKERS = _NUM_CORES * _NUM_SUBCORES
_LANES = 16

_VOCAB = 1000000
_BATCH = 4096
_SEQ = 200
_DIM = 64
_CHUNK = _SEQ
_T_PER_W = _BATCH * _SEQ // _NUM_WORKERS
_B_PER_W = _BATCH // _NUM_WORKERS
_H_PER_W = _T_PER_W // _CHUNK
_NBUF = 4
_LAG = _NBUF // 2
_NVEC = _DIM // _LANES


def _make_sc_call():
    mesh = plsc.VectorSubcoreMesh(core_axis_name="c", subcore_axis_name="s")

    @functools.partial(
        pl.kernel,
        mesh=mesh,
        compiler_params=pltpu.CompilerParams(use_tc_tiling_on_sc=False),
        out_type=jax.ShapeDtypeStruct((_BATCH, _DIM), jnp.float32),
        scratch_types=[
            pltpu.VMEM((_H_PER_W, _CHUNK), jnp.int32),
            pltpu.VMEM((_NBUF, _CHUNK, _DIM), jnp.float32),
            pltpu.VMEM((_NBUF, _CHUNK), jnp.int32),
            pltpu.VMEM((_B_PER_W, _DIM), jnp.float32),
            pltpu.VMEM_SHARED((_NUM_SUBCORES * _B_PER_W, _DIM), jnp.float32),
            [pltpu.SemaphoreType.DMA] * _NBUF,
            [pltpu.SemaphoreType.DMA] * _NBUF,
        ],
    )
    def enc(tokens_hbm, table_hbm, out_hbm, idx_v, rows_v, sidx_v, out_v,
            acc_sh, gsems, ssems):
        cid = lax.axis_index("c")
        sid = lax.axis_index("s")
        wid = sid * _NUM_CORES + cid
        base_h = wid * _H_PER_W
        base_b = wid * _B_PER_W
        own = sid * _B_PER_W

        zvec = jnp.zeros((_LANES,), jnp.float32)

        def zbody(r, carry):
            for k in range(_NVEC):
                out_v[r, pl.ds(k * _LANES, _LANES)] = zvec
            return carry

        lax.fori_loop(0, _B_PER_W, zbody, 0)
        pltpu.sync_copy(out_v, acc_sh.at[pl.ds(own, _B_PER_W)])

        pltpu.make_async_copy(
            tokens_hbm.at[pl.ds(base_h, _H_PER_W)], idx_v, gsems[0]).start()
        pltpu.make_async_copy(
            tokens_hbm.at[pl.ds(base_h, _H_PER_W)], idx_v, gsems[0]).wait()

        def gather_start(h, buf):
            def gbody(g, carry):
                vec = idx_v[h, pl.ds(g * _LANES, _LANES)]
                for l in range(_LANES):
                    r = g * _LANES + l
                    pltpu.make_async_copy(
                        table_hbm.at[vec[l]],
                        rows_v.at[buf, r],
                        gsems[buf]).start()
                return carry
            lax.fori_loop(0, 12, gbody, 0)
            vec = idx_v[h, pl.ds(_CHUNK - _LANES, _LANES)]
            for l in range(8, _LANES):
                r = _CHUNK - _LANES + l
                pltpu.make_async_copy(
                    table_hbm.at[vec[l]],
                    rows_v.at[buf, r],
                    gsems[buf]).start()

        def gather_wait(buf):
            pltpu.make_async_copy(
                table_hbm.at[pl.ds(0, _CHUNK)], rows_v.at[buf],
                gsems[buf]).wait()

        def scatter(buf):
            return pltpu.make_async_copy(
                rows_v.at[buf], acc_sh.at[sidx_v.at[buf]], ssems[buf])

        def scatter_start(buf):
            pltpu.async_copy(
                rows_v.at[buf], acc_sh.at[sidx_v.at[buf]], ssems[buf],
                add=True)

        _splat_offs = tuple(range(0, _CHUNK - _LANES, _LANES)) + (_CHUNK - _LANES,)

        def set_scatter_rows(buf, h):
            val = jnp.zeros((_LANES,), jnp.int32) + (own + h)
            for off in _splat_offs:
                sidx_v[buf, pl.ds(off, _LANES)] = val

        for b in range(_LAG):
            gather_start(jnp.int32(b), b)

        def outer(i, carry):
            for j in range(_NBUF):
                h = i * _NBUF + j
                gather_wait(j)
                set_scatter_rows(j, h)
                scatter_start(j)
                nb = (j + _LAG) % _NBUF
                nh = h + _LAG

                @pl.when(nh >= _NBUF)
                def _():
                    scatter(nb).wait()

                @pl.when(nh < _H_PER_W)
                def _():
                    gather_start(nh, nb)
            return carry

        lax.fori_loop(0, _H_PER_W // _NBUF, outer, 0)

        for j in range(_NBUF - _LAG, _NBUF):
            scatter(j).wait()

        pltpu.sync_copy(acc_sh.at[pl.ds(own, _B_PER_W)], out_v)
        inv_n = jnp.float32(1.0 / _SEQ)

        def scale(r, carry):
            for k in range(_NVEC):
                sl = pl.ds(k * _LANES, _LANES)
                out_v[r, sl] = out_v[r, sl] * inv_n
            return carry

        lax.fori_loop(0, _B_PER_W, scale, 0)
        pltpu.make_async_copy(
            out_v, out_hbm.at[pl.ds(base_b, _B_PER_W)], gsems[0]).start()
        pltpu.make_async_copy(
            out_v, out_hbm.at[pl.ds(base_b, _B_PER_W)], gsems[0]).wait()

    return enc


_sc_call = _make_sc_call()


def kernel(tokens, embedding_weight):
    table = jex_layout.with_layout_constraint(
        embedding_weight, jex_layout.Layout(major_to_minor=(0, 1)))
    return _sc_call(tokens, table)
